# baseline (speedup 1.0000x reference)
import functools

import jax
import jax.numpy as jnp
from jax import lax
from jax.experimental import pallas as pl
from jax.experimental.pallas import tpu as pltpu
from jax.experimental.pallas import tpu_sc as plsc

N = 10000
D = 128
E = 320000
NPAD = 10240
NC = 2
NS = 16
NW = NC * NS
EPW = 10752
E_EXT = NW * EPW
RPS = NPAD // NS

CD = 128
NCH_D = EPW // CD
CL = 112
NCH_L = EPW // CL

_mesh = plsc.VectorSubcoreMesh(core_axis_name="c", subcore_axis_name="s")
_params = pltpu.CompilerParams(needs_layout_passes=False)


@functools.partial(
    pl.kernel,
    mesh=_mesh,
    compiler_params=_params,
    out_type=jax.ShapeDtypeStruct((NC, NPAD), jnp.float32),
    scratch_types=[
        pltpu.VMEM((NCH_D, CD), jnp.int32),
        pltpu.VMEM((NCH_D, CD), jnp.float32),
        pltpu.VMEM((RPS,), jnp.float32),
        pltpu.VMEM_SHARED((NPAD,), jnp.float32),
        pltpu.SemaphoreType.DMA,
    ],
)
def _sc_deg(col_hbm, ew_hbm, degp_hbm, colv, ewv, zv, acc, sem):
    cid = lax.axis_index("c")
    sid = lax.axis_index("s")
    wid = cid * NS + sid

    z = jnp.zeros((16,), jnp.float32)

    def zbody(i, carry):
        zv[pl.ds(i * 16, 16)] = z
        return carry

    lax.fori_loop(0, RPS // 16, zbody, 0)
    pltpu.sync_copy(zv, acc.at[pl.ds(sid * RPS, RPS)])
    pltpu.sync_copy(col_hbm.at[wid], colv)
    pltpu.sync_copy(ew_hbm.at[wid], ewv)
    plsc.subcore_barrier()

    depth = 8
    for i in range(NCH_D):
        pltpu.async_copy(ewv.at[i], acc.at[colv.at[i]], sem, add=True)
        if i >= depth:
            j = i - depth
            pltpu.make_async_copy(ewv.at[j], acc.at[colv.at[j]], sem).wait()
    for j in range(NCH_D - depth, NCH_D):
        pltpu.make_async_copy(ewv.at[j], acc.at[colv.at[j]], sem).wait()

    plsc.subcore_barrier()
    pltpu.sync_copy(acc.at[pl.ds(sid * RPS, RPS)],
                    degp_hbm.at[cid, pl.ds(sid * RPS, RPS)])


def _scale_chunk(msg, ewv):
    def body(j, carry):
        n16 = ewv[pl.ds(j * 16, 16)]
        for e in range(16):
            s = n16[e]
            r = j * 16 + e
            for f in range(D // 16):
                sl = pl.ds(f * 16, 16)
                msg[r, sl] = msg[r, sl] * s
        return carry

    lax.fori_loop(0, CL // 16, body, 0)


@functools.partial(
    pl.kernel,
    mesh=_mesh,
    compiler_params=_params,
    out_type=jax.ShapeDtypeStruct((NC, NPAD, D), jnp.float32),
    scratch_types=[
        pltpu.VMEM((CL,), jnp.int32),
        pltpu.VMEM((CL,), jnp.int32),
        pltpu.VMEM((CL,), jnp.int32),
        pltpu.VMEM((CL,), jnp.int32),
        pltpu.VMEM((CL,), jnp.int32),
        pltpu.VMEM((CL,), jnp.int32),
        pltpu.VMEM((CL,), jnp.float32),
        pltpu.VMEM((CL,), jnp.float32),
        pltpu.VMEM((CL,), jnp.float32),
        pltpu.VMEM((CL, D), jnp.float32),
        pltpu.VMEM((CL, D), jnp.float32),
        pltpu.VMEM((CL, D), jnp.float32),
        pltpu.VMEM_SHARED((NPAD, D), jnp.float32),
    ] + [pltpu.SemaphoreType.DMA] * 15,
)
def _sc_layer(row_hbm, col_hbm, ew_hbm, g_hbm, part_hbm,
              rA, rB, rC, cA, cB, cC, eA, eB, eC, mA, mB, mC, acc,
              *sems):
    cid = lax.axis_index("c")
    sid = lax.axis_index("s")
    wid = cid * NS + sid

    rows = (rA, rB, rC)
    cols = (cA, cB, cC)
    ews = (eA, eB, eC)
    bufs = (mA, mB, mC)
    gsem = sems[0:3]
    ssem = sems[3:6]
    rsem = sems[6:9]
    csem = sems[9:12]
    esem = sems[12:15]

    def rowload(b, i):
        pltpu.async_copy(row_hbm.at[wid, i], rows[b], rsem[b])

    def wait_rowload(b, i):
        pltpu.make_async_copy(row_hbm.at[wid, i], rows[b], rsem[b]).wait()

    def colload(b, i):
        pltpu.async_copy(col_hbm.at[wid, i], cols[b], csem[b])

    def wait_colload(b, i):
        pltpu.make_async_copy(col_hbm.at[wid, i], cols[b], csem[b]).wait()

    def ewload(b, i):
        pltpu.async_copy(ew_hbm.at[wid, i], ews[b], esem[b])

    def wait_ewload(b, i):
        pltpu.make_async_copy(ew_hbm.at[wid, i], ews[b], esem[b]).wait()

    def gather(b, i):
        pltpu.async_copy(g_hbm.at[rows[b]], bufs[b], gsem[b])

    def wait_gather(b, i):
        pltpu.make_async_copy(g_hbm.at[rows[b]], bufs[b], gsem[b]).wait()

    def scatter(b, i):
        pltpu.async_copy(bufs[b], acc.at[cols[b]], ssem[b], add=True)

    def wait_scatter(b, i):
        pltpu.make_async_copy(bufs[b], acc.at[cols[b]], ssem[b]).wait()

    for b in range(3):
        rowload(b, b)
        colload(b, b)
        ewload(b, b)

    z = jnp.zeros((16,), jnp.float32)

    def zbody(e, carry):
        for f in range(D // 16):
            mA[e, pl.ds(f * 16, 16)] = z
        return carry

    lax.fori_loop(0, CL, zbody, 0)
    for k in range(RPS // CL):
        pltpu.sync_copy(mA, acc.at[pl.ds(sid * RPS + k * CL, CL)])
    rem = RPS - (RPS // CL) * CL
    if rem:
        pltpu.sync_copy(mA.at[pl.ds(0, rem)],
                        acc.at[pl.ds(sid * RPS + (RPS // CL) * CL, rem)])
    plsc.subcore_barrier()

    for b in range(3):
        wait_rowload(b, b)
        gather(b, b)

    def half(i, b, first=False, prefetch=True, issue=True):
        w = (b + 2) % 3
        wait_gather(b, i)
        if prefetch:
            rowload(b, i + 3)
        wait_ewload(b, i)
        _scale_chunk(bufs[b], ews[b])
        if prefetch:
            ewload(b, i + 3)
        if not first:
            wait_scatter(w, i - 1)
        if issue and not first:
            wait_rowload(w, i + 2)
            gather(w, i + 2)
            colload(w, i + 2)
        wait_colload(b, i)
        scatter(b, i)

    half(0, 0, first=True)
    half(1, 1)
    half(2, 2)

    def loop(k, carry):
        i0 = 3 * k
        for b in range(3):
            half(i0 + b, b)
        return carry

    lax.fori_loop(1, NCH_L // 3 - 1, loop, 0)

    i0 = NCH_L - 3
    half(i0, 0, prefetch=False)
    half(i0 + 1, 1, prefetch=False, issue=False)
    half(i0 + 2, 2, prefetch=False, issue=False)
    wait_scatter(2, i0 + 2)

    plsc.subcore_barrier()
    pltpu.sync_copy(acc.at[pl.ds(sid * RPS, RPS)],
                    part_hbm.at[cid, pl.ds(sid * RPS, RPS)])


def _tc_prep_body(x_ref, w1_ref, d0_ref, d1_ref, g_ref, dinvb_ref):
    dv = lax.rsqrt(d0_ref[...] + d1_ref[...])
    h = lax.dot_general(x_ref[...], w1_ref[...],
                        (((1,), (1,)), ((), ())),
                        preferred_element_type=jnp.float32)
    for s in range(8):
        dvt = jnp.transpose(
            jnp.broadcast_to(dv[s:s + 1, :], (128, 128)), (1, 0))
        rows = pl.ds(s * 128, 128)
        g_ref[rows, :] = h[s * 128:(s + 1) * 128, :] * dvt
        dinvb_ref[rows, :] = dvt


def _tc_prep(x_pad, w1, d0, d1):
    return pl.pallas_call(
        _tc_prep_body,
        grid=(_GRID,),
        in_specs=[
            pl.BlockSpec((_BLK, D), lambda i: (i, 0)),
            pl.BlockSpec((D, D), lambda i: (0, 0)),
            pl.BlockSpec((8, 128), lambda i: (i, 0)),
            pl.BlockSpec((8, 128), lambda i: (i, 0)),
        ],
        out_specs=[
            pl.BlockSpec((_BLK, D), lambda i: (i, 0)),
            pl.BlockSpec((_BLK, D), lambda i: (i, 0)),
        ],
        out_shape=[
            jax.ShapeDtypeStruct((NPAD, D), jnp.float32),
            jax.ShapeDtypeStruct((NPAD, D), jnp.float32),
        ],
    )(x_pad, w1, d0, d1)


_BLK = 1024
_GRID = NPAD // _BLK


def _tc_mid_body(p0_ref, p1_ref, db_ref, b_ref, w2_ref, g2_ref):
    db = db_ref[...]
    a1 = jax.nn.relu(db * (p0_ref[...] + p1_ref[...]) + b_ref[...])
    h2 = lax.dot_general(a1, w2_ref[...],
                         (((1,), (1,)), ((), ())),
                         preferred_element_type=jnp.float32)
    g2_ref[...] = h2 * db


def _tc_mid(p0, p1, dinvb, b1, w2):
    return pl.pallas_call(
        _tc_mid_body,
        grid=(_GRID,),
        in_specs=[
            pl.BlockSpec((_BLK, D), lambda i: (i, 0)),
            pl.BlockSpec((_BLK, D), lambda i: (i, 0)),
            pl.BlockSpec((_BLK, D), lambda i: (i, 0)),
            pl.BlockSpec((1, D), lambda i: (0, 0)),
            pl.BlockSpec((D, D), lambda i: (0, 0)),
        ],
        out_specs=pl.BlockSpec((_BLK, D), lambda i: (i, 0)),
        out_shape=jax.ShapeDtypeStruct((NPAD, D), jnp.float32),
    )(p0, p1, dinvb, b1.reshape(1, D), w2)


def _tc_final_body(p0_ref, p1_ref, db_ref, b_ref, out_ref):
    out_ref[...] = jax.nn.relu(
        db_ref[...] * (p0_ref[...] + p1_ref[...]) + b_ref[...])


def _tc_final(p0, p1, dinvb, b2):
    return pl.pallas_call(
        _tc_final_body,
        grid=(_GRID,),
        in_specs=[
            pl.BlockSpec((_BLK, D), lambda i: (i, 0)),
            pl.BlockSpec((_BLK, D), lambda i: (i, 0)),
            pl.BlockSpec((_BLK, D), lambda i: (i, 0)),
            pl.BlockSpec((1, D), lambda i: (0, 0)),
        ],
        out_specs=pl.BlockSpec((_BLK, D), lambda i: (i, 0)),
        out_shape=jax.ShapeDtypeStruct((NPAD, D), jnp.float32),
    )(p0, p1, dinvb, b2.reshape(1, D))


def kernel(x, edge_index, edge_weights, W1, b1, W2, b2):
    row = edge_index[0]
    col = edge_index[1]
    sl = jnp.arange(NPAD, dtype=jnp.int32)
    npad_e = E_EXT - E - NPAD
    pad_i = jnp.full((npad_e,), NPAD - 1, jnp.int32)
    row_ext = jnp.concatenate([row, sl, pad_i])
    col_ext = jnp.concatenate([col, sl, pad_i])
    ew_ext = jnp.concatenate([edge_weights,
                              jnp.ones((NPAD,), jnp.float32),
                              jnp.zeros((npad_e,), jnp.float32)])
    x_pad = jnp.pad(x, ((0, NPAD - N), (0, 0)))

    col_d = col_ext.reshape(NW, NCH_D, CD)
    ew_d = ew_ext.reshape(NW, NCH_D, CD)
    row_l = row_ext.reshape(NW, NCH_L, CL)
    col_l = col_ext.reshape(NW, NCH_L, CL)
    ew_l = ew_ext.reshape(NW, NCH_L, CL)

    degp = _sc_deg(col_d, ew_d)
    d0 = degp[0].reshape(NPAD // 128, 128)
    d1 = degp[1].reshape(NPAD // 128, 128)
    g1, dinvb = _tc_prep(x_pad, W1, d0, d1)
    part1 = _sc_layer(row_l, col_l, ew_l, g1)
    g2 = _tc_mid(part1[0], part1[1], dinvb, b1, W2)
    part2 = _sc_layer(row_l, col_l, ew_l, g2)
    out = _tc_final(part2[0], part2[1], dinvb, b2)
    return out[:N]


# speedup vs baseline: 5.3515x; 5.3515x over previous
import functools

import jax
import jax.numpy as jnp
from jax import lax
from jax.experimental import pallas as pl
from jax.experimental.pallas import tpu as pltpu
from jax.experimental.pallas import tpu_sc as plsc

N = 10000
D = 128
E = 320000
NPAD = 10240
NC = 2
NS = 16
NW = NC * NS
EPW = 10752
E_EXT = NW * EPW
RPS = NPAD // NS

CD = 128
NCH_D = EPW // CD
CL = 112
NCH_L = EPW // CL

_mesh = plsc.VectorSubcoreMesh(core_axis_name="c", subcore_axis_name="s")
_params = pltpu.CompilerParams(needs_layout_passes=False)


@functools.partial(
    pl.kernel,
    mesh=_mesh,
    compiler_params=_params,
    out_type=jax.ShapeDtypeStruct((NC, NPAD), jnp.float32),
    scratch_types=[
        pltpu.VMEM((NCH_D, CD), jnp.int32),
        pltpu.VMEM((NCH_D, CD), jnp.float32),
        pltpu.VMEM((RPS,), jnp.float32),
        pltpu.VMEM_SHARED((NPAD,), jnp.float32),
        pltpu.SemaphoreType.DMA,
    ],
)
def _sc_deg(col_hbm, ew_hbm, degp_hbm, colv, ewv, zv, acc, sem):
    cid = lax.axis_index("c")
    sid = lax.axis_index("s")
    wid = cid * NS + sid

    z = jnp.zeros((16,), jnp.float32)

    def zbody(i, carry):
        zv[pl.ds(i * 16, 16)] = z
        return carry

    lax.fori_loop(0, RPS // 16, zbody, 0)
    pltpu.sync_copy(zv, acc.at[pl.ds(sid * RPS, RPS)])
    pltpu.sync_copy(col_hbm.at[wid], colv)
    pltpu.sync_copy(ew_hbm.at[wid], ewv)
    plsc.subcore_barrier()

    depth = 8
    for i in range(NCH_D):
        pltpu.async_copy(ewv.at[i], acc.at[colv.at[i]], sem, add=True)
        if i >= depth:
            j = i - depth
            pltpu.make_async_copy(ewv.at[j], acc.at[colv.at[j]], sem).wait()
    for j in range(NCH_D - depth, NCH_D):
        pltpu.make_async_copy(ewv.at[j], acc.at[colv.at[j]], sem).wait()

    plsc.subcore_barrier()
    pltpu.sync_copy(acc.at[pl.ds(sid * RPS, RPS)],
                    degp_hbm.at[cid, pl.ds(sid * RPS, RPS)])


def _scale_chunk(msg, ewv):
    def body(j, carry):
        n16 = ewv[pl.ds(j * 16, 16)]
        for e in range(16):
            s = n16[e]
            r = j * 16 + e
            for f in range(D // 16):
                sl = pl.ds(f * 16, 16)
                msg[r, sl] = msg[r, sl] * s
        return carry

    lax.fori_loop(0, CL // 16, body, 0)


@functools.partial(
    pl.kernel,
    mesh=_mesh,
    compiler_params=_params,
    out_type=jax.ShapeDtypeStruct((NC, NPAD, D), jnp.float32),
    scratch_types=[
        pltpu.VMEM((CL,), jnp.int32),
        pltpu.VMEM((CL,), jnp.int32),
        pltpu.VMEM((CL,), jnp.int32),
        pltpu.VMEM((CL,), jnp.int32),
        pltpu.VMEM((CL,), jnp.int32),
        pltpu.VMEM((CL,), jnp.int32),
        pltpu.VMEM((CL,), jnp.float32),
        pltpu.VMEM((CL,), jnp.float32),
        pltpu.VMEM((CL,), jnp.float32),
        pltpu.VMEM((CL, D), jnp.float32),
        pltpu.VMEM((CL, D), jnp.float32),
        pltpu.VMEM((CL, D), jnp.float32),
        pltpu.VMEM_SHARED((NPAD, D), jnp.float32),
    ] + [pltpu.SemaphoreType.DMA] * 15,
)
def _sc_layer(row_hbm, col_hbm, ew_hbm, g_hbm, part_hbm,
              rA, rB, rC, cA, cB, cC, eA, eB, eC, mA, mB, mC, acc,
              *sems):
    cid = lax.axis_index("c")
    sid = lax.axis_index("s")
    wid = cid * NS + sid

    rows = (rA, rB, rC)
    cols = (cA, cB, cC)
    ews = (eA, eB, eC)
    bufs = (mA, mB, mC)
    gsem = sems[0:3]
    ssem = sems[3:6]
    rsem = sems[6:9]
    csem = sems[9:12]
    esem = sems[12:15]

    def rowload(b, i):
        pltpu.async_copy(row_hbm.at[wid, i], rows[b], rsem[b])

    def wait_rowload(b, i):
        pltpu.make_async_copy(row_hbm.at[wid, i], rows[b], rsem[b]).wait()

    def colload(b, i):
        pltpu.async_copy(col_hbm.at[wid, i], cols[b], csem[b])

    def wait_colload(b, i):
        pltpu.make_async_copy(col_hbm.at[wid, i], cols[b], csem[b]).wait()

    def ewload(b, i):
        pltpu.async_copy(ew_hbm.at[wid, i], ews[b], esem[b])

    def wait_ewload(b, i):
        pltpu.make_async_copy(ew_hbm.at[wid, i], ews[b], esem[b]).wait()

    def gather(b, i):
        pltpu.async_copy(g_hbm.at[rows[b]], bufs[b], gsem[b])

    def wait_gather(b, i):
        pltpu.make_async_copy(g_hbm.at[rows[b]], bufs[b], gsem[b]).wait()

    def scatter(b, i):
        pltpu.async_copy(bufs[b], acc.at[cols[b]], ssem[b], add=True)

    def wait_scatter(b, i):
        pltpu.make_async_copy(bufs[b], acc.at[cols[b]], ssem[b]).wait()

    for b in range(3):
        rowload(b, b)
        colload(b, b)
        ewload(b, b)

    z = jnp.zeros((16,), jnp.float32)

    def zbody(e, carry):
        for f in range(D // 16):
            mA[e, pl.ds(f * 16, 16)] = z
        return carry

    lax.fori_loop(0, CL, zbody, 0)
    for k in range(RPS // CL):
        pltpu.sync_copy(mA, acc.at[pl.ds(sid * RPS + k * CL, CL)])
    rem = RPS - (RPS // CL) * CL
    if rem:
        pltpu.sync_copy(mA.at[pl.ds(0, rem)],
                        acc.at[pl.ds(sid * RPS + (RPS // CL) * CL, rem)])
    plsc.subcore_barrier()

    for b in range(3):
        wait_rowload(b, b)
        gather(b, b)

    def half(i, b, first=False, prefetch=True, issue=True):
        w = (b + 2) % 3
        wait_gather(b, i)
        if prefetch:
            rowload(b, i + 3)
        wait_ewload(b, i)
        _scale_chunk(bufs[b], ews[b])
        if prefetch:
            ewload(b, i + 3)
        if not first:
            wait_scatter(w, i - 1)
        if issue and not first:
            wait_rowload(w, i + 2)
            gather(w, i + 2)
            colload(w, i + 2)
        wait_colload(b, i)
        scatter(b, i)

    half(0, 0, first=True)
    half(1, 1)
    half(2, 2)

    def loop(k, carry):
        i0 = 3 * k
        for b in range(3):
            half(i0 + b, b)
        return carry

    lax.fori_loop(1, NCH_L // 3 - 1, loop, 0)

    i0 = NCH_L - 3
    half(i0, 0, prefetch=False)
    half(i0 + 1, 1, prefetch=False, issue=False)
    half(i0 + 2, 2, prefetch=False, issue=False)
    wait_scatter(2, i0 + 2)

    plsc.subcore_barrier()
    pltpu.sync_copy(acc.at[pl.ds(sid * RPS, RPS)],
                    part_hbm.at[cid, pl.ds(sid * RPS, RPS)])


def _tc_prep_body(x_ref, w1_ref, d0_ref, d1_ref, g_ref, dinvb_ref):
    dv = lax.rsqrt(d0_ref[...] + d1_ref[...])
    h = lax.dot_general(x_ref[...], w1_ref[...],
                        (((1,), (1,)), ((), ())),
                        preferred_element_type=jnp.float32)
    for s in range(8):
        dvt = jnp.transpose(
            jnp.broadcast_to(dv[s:s + 1, :], (128, 128)), (1, 0))
        rows = pl.ds(s * 128, 128)
        g_ref[rows, :] = h[s * 128:(s + 1) * 128, :] * dvt
        dinvb_ref[rows, :] = dvt


def _tc_prep(x_pad, w1, d0, d1):
    return pl.pallas_call(
        _tc_prep_body,
        grid=(_GRID,),
        in_specs=[
            pl.BlockSpec((_BLK, D), lambda i: (i, 0)),
            pl.BlockSpec((D, D), lambda i: (0, 0)),
            pl.BlockSpec((8, 128), lambda i: (i, 0)),
            pl.BlockSpec((8, 128), lambda i: (i, 0)),
        ],
        out_specs=[
            pl.BlockSpec((_BLK, D), lambda i: (i, 0)),
            pl.BlockSpec((_BLK, D), lambda i: (i, 0)),
        ],
        out_shape=[
            jax.ShapeDtypeStruct((NPAD, D), jnp.float32),
            jax.ShapeDtypeStruct((NPAD, D), jnp.float32),
        ],
    )(x_pad, w1, d0, d1)


_BLK = 1024
_GRID = NPAD // _BLK


def _tc_mid_body(p0_ref, p1_ref, db_ref, b_ref, w2_ref, g2_ref):
    db = db_ref[...]
    a1 = jax.nn.relu(db * (p0_ref[...] + p1_ref[...]) + b_ref[...])
    h2 = lax.dot_general(a1, w2_ref[...],
                         (((1,), (1,)), ((), ())),
                         preferred_element_type=jnp.float32)
    g2_ref[...] = h2 * db


def _tc_mid(p0, p1, dinvb, b1, w2):
    return pl.pallas_call(
        _tc_mid_body,
        grid=(_GRID,),
        in_specs=[
            pl.BlockSpec((_BLK, D), lambda i: (i, 0)),
            pl.BlockSpec((_BLK, D), lambda i: (i, 0)),
            pl.BlockSpec((_BLK, D), lambda i: (i, 0)),
            pl.BlockSpec((1, D), lambda i: (0, 0)),
            pl.BlockSpec((D, D), lambda i: (0, 0)),
        ],
        out_specs=pl.BlockSpec((_BLK, D), lambda i: (i, 0)),
        out_shape=jax.ShapeDtypeStruct((NPAD, D), jnp.float32),
    )(p0, p1, dinvb, b1.reshape(1, D), w2)


def _tc_final_body(p0_ref, p1_ref, db_ref, b_ref, out_ref):
    out_ref[...] = jax.nn.relu(
        db_ref[...] * (p0_ref[...] + p1_ref[...]) + b_ref[...])


def _tc_final(p0, p1, dinvb, b2):
    return pl.pallas_call(
        _tc_final_body,
        grid=(_GRID,),
        in_specs=[
            pl.BlockSpec((_BLK, D), lambda i: (i, 0)),
            pl.BlockSpec((_BLK, D), lambda i: (i, 0)),
            pl.BlockSpec((_BLK, D), lambda i: (i, 0)),
            pl.BlockSpec((1, D), lambda i: (0, 0)),
        ],
        out_specs=pl.BlockSpec((_BLK, D), lambda i: (i, 0)),
        out_shape=jax.ShapeDtypeStruct((NPAD, D), jnp.float32),
    )(p0, p1, dinvb, b2.reshape(1, D))


def kernel(x, edge_index, edge_weights, W1, b1, W2, b2):
    row = edge_index[0]
    col = edge_index[1]
    sl = jnp.arange(NPAD, dtype=jnp.int32)
    npad_e = E_EXT - E - NPAD
    pad_i = (jnp.arange(npad_e, dtype=jnp.int32) * 37) % NPAD
    row_ext = jnp.concatenate([row, sl, pad_i])
    col_ext = jnp.concatenate([col, sl, pad_i])
    ew_ext = jnp.concatenate([edge_weights,
                              jnp.ones((NPAD,), jnp.float32),
                              jnp.zeros((npad_e,), jnp.float32)])
    x_pad = jnp.pad(x, ((0, NPAD - N), (0, 0)))

    col_d = col_ext.reshape(NW, NCH_D, CD)
    ew_d = ew_ext.reshape(NW, NCH_D, CD)
    row_l = row_ext.reshape(NW, NCH_L, CL)
    col_l = col_ext.reshape(NW, NCH_L, CL)
    ew_l = ew_ext.reshape(NW, NCH_L, CL)

    degp = _sc_deg(col_d, ew_d)
    d0 = degp[0].reshape(NPAD // 128, 128)
    d1 = degp[1].reshape(NPAD // 128, 128)
    g1, dinvb = _tc_prep(x_pad, W1, d0, d1)
    part1 = _sc_layer(row_l, col_l, ew_l, g1)
    g2 = _tc_mid(part1[0], part1[1], dinvb, b1, W2)
    part2 = _sc_layer(row_l, col_l, ew_l, g2)
    out = _tc_final(part2[0], part2[1], dinvb, b2)
    return out[:N]
